# baseline (speedup 1.0000x reference)
import functools

import jax
import jax.numpy as jnp
from jax import lax
from jax.experimental import pallas as pl
from jax.experimental.pallas import tpu as pltpu
from jax.experimental.pallas import tpu_sc as plsc

D = 64
L = 16
NC = 2
NS = 16
NW = NC * NS
B = 4096 * 200
ROWS_PER_W = B // NW
CHUNK = 128
NCHUNK = ROWS_PER_W // CHUNK
SCALE = 8.0

_mesh = plsc.VectorSubcoreMesh(core_axis_name="c", subcore_axis_name="s")


@functools.partial(
    pl.kernel,
    mesh=_mesh,
    out_type=jax.ShapeDtypeStruct((B, D), jnp.float32),
    scratch_types=[
        pltpu.VMEM((NCHUNK, CHUNK), jnp.int32),
        pltpu.VMEM((CHUNK, D), jnp.float32),
        pltpu.SemaphoreType.DMA,
    ],
    compiler_params=pltpu.CompilerParams(use_tc_tiling_on_sc=False),
)
def _emb(table_hbm, tok_hbm, out_hbm, idx_v, rows_v, sem):
    wid = lax.axis_index("s") * NC + lax.axis_index("c")
    base = wid * ROWS_PER_W
    pltpu.sync_copy(tok_hbm.at[pl.ds(wid * NCHUNK, NCHUNK)], idx_v)

    def chunk_body(g, carry):
        pltpu.async_copy(table_hbm.at[idx_v.at[g]], rows_v, sem).wait()

        def scale_row(i, c):
            for j in range(D // L):
                rows_v[i, pl.ds(j * L, L)] = rows_v[i, pl.ds(j * L, L)] * SCALE
            return c

        lax.fori_loop(0, CHUNK, scale_row, 0)
        pltpu.sync_copy(rows_v, out_hbm.at[pl.ds(base + g * CHUNK, CHUNK)])
        return carry

    lax.fori_loop(0, NCHUNK, chunk_body, 0)


def kernel(tokens, table):
    nb, nt = tokens.shape
    tok = tokens.astype(jnp.int32).reshape(NW * NCHUNK, CHUNK)
    out = _emb(table, tok)
    return out.reshape(nb, nt, D)


# speedup vs baseline: 1.2118x; 1.2118x over previous
import functools

import jax
import jax.numpy as jnp
from jax import lax
from jax.experimental import pallas as pl
from jax.experimental.pallas import tpu as pltpu
from jax.experimental.pallas import tpu_sc as plsc

D = 64
L = 16
NC = 2
NS = 16
NW = NC * NS
B = 4096 * 200
ROWS_PER_W = B // NW
CHUNK = 128
NCHUNK = ROWS_PER_W // CHUNK
NBUF = 4
T = NCHUNK // NBUF
SCALE = 8.0

_mesh = plsc.VectorSubcoreMesh(core_axis_name="c", subcore_axis_name="s")


@functools.partial(
    pl.kernel,
    mesh=_mesh,
    out_type=jax.ShapeDtypeStruct((B, D), jnp.float32),
    scratch_types=[
        pltpu.VMEM((NCHUNK, CHUNK), jnp.int32),
        pltpu.VMEM((NBUF, CHUNK, D), jnp.float32),
        pltpu.VMEM((NBUF, CHUNK, D), jnp.float32),
        pltpu.SemaphoreType.DMA((NBUF,)),
        pltpu.SemaphoreType.DMA((NBUF,)),
    ],
    compiler_params=pltpu.CompilerParams(use_tc_tiling_on_sc=False),
)
def _emb(table_hbm, tok_hbm, out_hbm, idx_v, rin, rout, sem_g, sem_w):
    wid = lax.axis_index("s") * NC + lax.axis_index("c")
    base = wid * ROWS_PER_W
    pltpu.sync_copy(tok_hbm.at[pl.ds(wid * NCHUNK, NCHUNK)], idx_v)

    def start_gather(g, b):
        pltpu.make_async_copy(
            table_hbm.at[idx_v.at[g]], rin.at[b], sem_g.at[b]).start()

    def wait_gather(b):
        pltpu.make_async_copy(
            table_hbm.at[pl.ds(0, CHUNK)], rin.at[b], sem_g.at[b]).wait()

    def start_wb(g, b):
        pltpu.make_async_copy(
            rout.at[b], out_hbm.at[pl.ds(base + g * CHUNK, CHUNK)],
            sem_w.at[b]).start()

    def wait_wb(b):
        pltpu.make_async_copy(
            rout.at[b], out_hbm.at[pl.ds(base, CHUNK)], sem_w.at[b]).wait()

    def scale(b):
        def srow(i, c):
            for j in range(D // L):
                rout[b, i, pl.ds(j * L, L)] = rin[b, i, pl.ds(j * L, L)] * SCALE
            return c
        lax.fori_loop(0, CHUNK, srow, 0)

    for b in range(NBUF):
        start_gather(b, b)

    for b in range(NBUF):
        wait_gather(b)
        scale(b)
        start_wb(b, b)
        start_gather(b + NBUF, b)

    def mid(t, c):
        for b in range(NBUF):
            g = t * NBUF + b
            wait_gather(b)
            wait_wb(b)
            scale(b)
            start_wb(g, b)
            start_gather(g + NBUF, b)
        return c

    lax.fori_loop(1, T - 1, mid, 0)

    for b in range(NBUF):
        g = (T - 1) * NBUF + b
        wait_gather(b)
        wait_wb(b)
        scale(b)
        start_wb(g, b)

    for b in range(NBUF):
        wait_wb(b)


def kernel(tokens, table):
    nb, nt = tokens.shape
    tok = tokens.astype(jnp.int32).reshape(NW * NCHUNK, CHUNK)
    out = _emb(table, tok)
    return out.reshape(nb, nt, D)
